# baseline (speedup 1.0000x reference)
import functools

import jax
import jax.numpy as jnp
from jax import lax
from jax.experimental import pallas as pl
from jax.experimental.pallas import tpu as pltpu
from jax.experimental.pallas import tpu_sc as plsc

_B, _H, _W = 16, 512, 512
_CARD = 50
_N = _B * _CARD * 2
_PER_TILE = 64
_ACTIVE = _N // _PER_TILE
_LANES = 16


def _sc_gather(x2d, packed):
    mesh = plsc.VectorSubcoreMesh(core_axis_name="c", subcore_axis_name="s")
    info = plsc.get_sparse_core_info()
    num_cores = info.num_cores

    @functools.partial(
        pl.kernel,
        mesh=mesh,
        out_type=jax.ShapeDtypeStruct((_N,), jnp.float32),
        scratch_types=[
            pltpu.VMEM((_PER_TILE * 8, 128), jnp.float32),
            pltpu.VMEM((_PER_TILE,), jnp.int32),
            pltpu.VMEM((_PER_TILE,), jnp.float32),
            pltpu.SemaphoreType.DMA,
        ],
    )
    def body(x_hbm, p_hbm, out_hbm, win_v, p_v, vals_v, sem):
        wid = lax.axis_index("s") * num_cores + lax.axis_index("c")

        @pl.when(wid < _ACTIVE)
        def _():
            base = wid * _PER_TILE
            pltpu.sync_copy(p_hbm.at[pl.ds(base, _PER_TILE)], p_v)
            lanes = lax.iota(jnp.int32, _LANES)
            scalars = []
            for g in range(_PER_TILE // _LANES):
                w16 = p_v[pl.ds(g * _LANES, _LANES)]
                for l in range(_LANES):
                    w = w16[l]
                    r = w >> 10
                    c = (w >> 1) & jnp.int32(_W - 1)
                    scalars.append((r, c))
                    j = g * _LANES + l
                    r8 = pl.multiple_of(r & ~7, 8)
                    c128 = pl.multiple_of(c & ~127, 128)
                    pltpu.async_copy(
                        x_hbm.at[pl.ds(r8, 8), pl.ds(c128, 128)],
                        win_v.at[pl.ds(j * 8, 8), pl.ds(0, 128)],
                        sem,
                    )
            for j in range(_PER_TILE):
                pltpu.make_async_copy(
                    x_hbm.at[pl.ds(0, 8), pl.ds(0, 128)],
                    win_v.at[pl.ds(j * 8, 8), pl.ds(0, 128)],
                    sem,
                ).wait()
            for g in range(_PER_TILE // _LANES):
                s = pl.ds(g * _LANES, _LANES)
                acc = jnp.zeros((_LANES,), jnp.float32)
                for l in range(_LANES):
                    j = g * _LANES + l
                    r, c = scalars[j]
                    c16 = pl.multiple_of((c & 127) & ~15, 16)
                    lane = c & 15
                    v16 = win_v[jnp.int32(j * 8) + (r & 7),
                                pl.ds(c16, _LANES)]
                    val = v16[0]
                    for k in range(1, _LANES):
                        val = jnp.where(lane == k, v16[k], val)
                    acc = jnp.where(lanes == l, val, acc)
                vals_v[s] = jnp.where((p_v[s] & 1) != 0, jnp.float32(0.0),
                                      acc)
            pltpu.sync_copy(vals_v, out_hbm.at[pl.ds(base, _PER_TILE)])

    return body(x2d, packed)


def kernel(x, ids0, ids1, ids_mask):
    x2d = x.reshape(_B * _H, _W)
    packed = (
        (ids0 << 10) | (ids1 << 1) | ids_mask.astype(jnp.int32)
    ).reshape(_N)
    flat = _sc_gather(x2d, packed)
    return flat.reshape(_B, _CARD, 2)
